# Initial kernel scaffold; baseline (speedup 1.0000x reference)
import jax
import jax.numpy as jnp
from jax.experimental import pallas as pl


def kernel(pc_gt, pc_pred):
    raise NotImplementedError("write your pallas kernel here")


# speedup vs baseline: 1.8362x; 1.8362x over previous
import functools

import jax
import jax.numpy as jnp
from jax.experimental import pallas as pl
from jax.experimental.pallas import tpu as pltpu

_B = 4
_N = 4096
_M = 4096
_BM = 512


def _chamfer_body(pred_t_ref, gt_t_ref, pred_r_ref, out_ref, colmin_ref):
    b = pl.program_id(0)
    i = pl.program_id(1)
    n_i = pl.num_programs(1)

    pred_t = pred_t_ref[0]
    gt_t = gt_t_ref[0]
    pred_r = pred_r_ref[0]

    g = jax.lax.dot_general(
        pred_t * (-2.0), gt_t,
        (((0,), (0,)), ((), ())),
        preferred_element_type=jnp.float32,
        precision=jax.lax.Precision.HIGHEST,
    )

    x2c = jnp.sum(pred_r * pred_r, axis=1, keepdims=True)
    y2r = jnp.sum(gt_t * gt_t, axis=0, keepdims=True)

    @pl.when(jnp.logical_and(b == 0, i == 0))
    def _init_out():
        out_ref[...] = jnp.zeros((1, 1), jnp.float32)

    @pl.when(i == 0)
    def _init_colmin():
        colmin_ref[...] = jnp.full(colmin_ref.shape, jnp.inf, jnp.float32)

    row_min = jnp.min(g + y2r, axis=1, keepdims=True)
    out_ref[...] += (jnp.sum(row_min, axis=(0, 1), keepdims=True)
                     + jnp.sum(x2c, axis=(0, 1), keepdims=True))

    colmin_ref[...] = jnp.minimum(
        colmin_ref[...], jnp.min(g + x2c, axis=0, keepdims=True))

    @pl.when(i == n_i - 1)
    def _finish_cols():
        out_ref[...] += jnp.sum(colmin_ref[...] + y2r, axis=(0, 1),
                                keepdims=True)


@jax.jit
def kernel(pc_gt, pc_pred):
    B, N, D = pc_gt.shape
    M = pc_pred.shape[1]

    pred_t = jnp.zeros((B, 8, M), jnp.float32).at[:, :D, :].set(
        pc_pred.transpose(0, 2, 1))
    gt_t = jnp.zeros((B, 8, N), jnp.float32).at[:, :D, :].set(
        pc_gt.transpose(0, 2, 1))
    pred_r = jnp.zeros((B, M, 8), jnp.float32).at[:, :, :D].set(pc_pred)

    total = pl.pallas_call(
        _chamfer_body,
        grid=(B, M // _BM),
        in_specs=[
            pl.BlockSpec((1, 8, _BM), lambda b, i: (b, 0, i)),
            pl.BlockSpec((1, 8, N), lambda b, i: (b, 0, 0)),
            pl.BlockSpec((1, _BM, 8), lambda b, i: (b, i, 0)),
        ],
        out_specs=pl.BlockSpec((1, 1), lambda b, i: (0, 0)),
        out_shape=jax.ShapeDtypeStruct((1, 1), jnp.float32),
        scratch_shapes=[pltpu.VMEM((1, N), jnp.float32)],
    )(pred_t, gt_t, pred_r)

    return (total[0, 0] / (B * M)).astype(jnp.float32)


# speedup vs baseline: 2.0240x; 1.1023x over previous
import jax
import jax.numpy as jnp
from jax.experimental import pallas as pl
from jax.experimental.pallas import tpu as pltpu

_BM = 512


def _chamfer_body(pred_a_ref, gt_a_ref, out_ref, colmin_ref):
    b = pl.program_id(0)
    i = pl.program_id(1)
    n_i = pl.num_programs(1)

    pred_a = pred_a_ref[0]
    gt_a = gt_a_ref[0]

    d2 = jax.lax.dot_general(
        pred_a, gt_a,
        (((0,), (0,)), ((), ())),
        preferred_element_type=jnp.float32,
        precision=jax.lax.Precision.HIGHEST,
    )

    @pl.when(jnp.logical_and(b == 0, i == 0))
    def _init_out():
        out_ref[...] = jnp.zeros((1, 1), jnp.float32)

    @pl.when(i == 0)
    def _init_colmin():
        colmin_ref[...] = jnp.full(colmin_ref.shape, jnp.inf, jnp.float32)

    row_min = jnp.min(d2, axis=1, keepdims=True)
    out_ref[...] += jnp.sum(row_min, axis=(0, 1), keepdims=True)

    colmin_ref[...] = jnp.minimum(
        colmin_ref[...], jnp.min(d2, axis=0, keepdims=True))

    @pl.when(i == n_i - 1)
    def _finish_cols():
        out_ref[...] += jnp.sum(colmin_ref[...], axis=(0, 1), keepdims=True)


@jax.jit
def kernel(pc_gt, pc_pred):
    B, N, D = pc_gt.shape
    M = pc_pred.shape[1]

    x2 = jnp.sum(pc_pred * pc_pred, axis=2)
    y2 = jnp.sum(pc_gt * pc_gt, axis=2)
    ones_m = jnp.ones((B, 1, M), jnp.float32)
    ones_n = jnp.ones((B, 1, N), jnp.float32)
    zeros_m = jnp.zeros((B, 3, M), jnp.float32)
    zeros_n = jnp.zeros((B, 3, N), jnp.float32)
    pred_a = jnp.concatenate(
        [pc_pred.transpose(0, 2, 1) * (-2.0), x2[:, None, :], ones_m,
         zeros_m], axis=1)
    gt_a = jnp.concatenate(
        [pc_gt.transpose(0, 2, 1), ones_n, y2[:, None, :], zeros_n],
        axis=1)

    total = pl.pallas_call(
        _chamfer_body,
        grid=(B, M // _BM),
        in_specs=[
            pl.BlockSpec((1, 8, _BM), lambda b, i: (b, 0, i)),
            pl.BlockSpec((1, 8, N), lambda b, i: (b, 0, 0)),
        ],
        out_specs=pl.BlockSpec((1, 1), lambda b, i: (0, 0)),
        out_shape=jax.ShapeDtypeStruct((1, 1), jnp.float32),
        scratch_shapes=[pltpu.VMEM((1, N), jnp.float32)],
    )(pred_a, gt_a)

    return (total[0, 0] / (B * M)).astype(jnp.float32)


# speedup vs baseline: 6.5356x; 3.2290x over previous
import jax
import jax.numpy as jnp
from jax.experimental import pallas as pl
from jax.experimental.pallas import tpu as pltpu

_BM = 512


def _chamfer_body(pred_a_ref, gt_a_ref, out_ref, colmin_ref):
    b = pl.program_id(0)
    i = pl.program_id(1)
    n_i = pl.num_programs(1)

    pred_a = pred_a_ref[0]
    gt_a = gt_a_ref[0]

    d2 = jax.lax.dot_general(
        pred_a, gt_a,
        (((0,), (0,)), ((), ())),
        preferred_element_type=jnp.float32,
    )

    @pl.when(jnp.logical_and(b == 0, i == 0))
    def _init_out():
        out_ref[...] = jnp.zeros((1, 1), jnp.float32)

    @pl.when(i == 0)
    def _init_colmin():
        colmin_ref[...] = jnp.full(colmin_ref.shape, jnp.inf, jnp.float32)

    row_min = jnp.min(d2, axis=1, keepdims=True)
    out_ref[...] += jnp.sum(row_min, axis=(0, 1), keepdims=True)

    colmin_ref[...] = jnp.minimum(
        colmin_ref[...], jnp.min(d2, axis=0, keepdims=True))

    @pl.when(i == n_i - 1)
    def _finish_cols():
        out_ref[...] += jnp.sum(colmin_ref[...], axis=(0, 1), keepdims=True)


@jax.jit
def kernel(pc_gt, pc_pred):
    B, N, D = pc_gt.shape
    M = pc_pred.shape[1]

    x2 = jnp.sum(pc_pred * pc_pred, axis=2)
    y2 = jnp.sum(pc_gt * pc_gt, axis=2)
    ones_m = jnp.ones((B, 1, M), jnp.float32)
    ones_n = jnp.ones((B, 1, N), jnp.float32)
    zeros_m = jnp.zeros((B, 3, M), jnp.float32)
    zeros_n = jnp.zeros((B, 3, N), jnp.float32)
    pred_a = jnp.concatenate(
        [pc_pred.transpose(0, 2, 1) * (-2.0), x2[:, None, :], ones_m,
         zeros_m], axis=1)
    gt_a = jnp.concatenate(
        [pc_gt.transpose(0, 2, 1), ones_n, y2[:, None, :], zeros_n],
        axis=1)

    def _hilo(a):
        hi = a.astype(jnp.bfloat16)
        lo = (a - hi.astype(jnp.float32)).astype(jnp.bfloat16)
        return hi, lo

    pah, pal = _hilo(pred_a)
    gah, gal = _hilo(gt_a)
    pred_k32 = jnp.concatenate([pah, pal, pah, pal], axis=1)
    gt_k32 = jnp.concatenate([gah, gah, gal, gal], axis=1)

    total = pl.pallas_call(
        _chamfer_body,
        grid=(B, M // _BM),
        in_specs=[
            pl.BlockSpec((1, 32, _BM), lambda b, i: (b, 0, i)),
            pl.BlockSpec((1, 32, N), lambda b, i: (b, 0, 0)),
        ],
        out_specs=pl.BlockSpec((1, 1), lambda b, i: (0, 0)),
        out_shape=jax.ShapeDtypeStruct((1, 1), jnp.float32),
        scratch_shapes=[pltpu.VMEM((1, N), jnp.float32)],
    )(pred_k32, gt_k32)

    return (total[0, 0] / (B * M)).astype(jnp.float32)
